# baseline (speedup 1.0000x reference)
import functools

import jax
import jax.numpy as jnp
from jax import lax
from jax.experimental import pallas as pl
from jax.experimental.pallas import tpu as pltpu
from jax.experimental.pallas import tpu_sc as plsc

_HIGH = jax.lax.Precision.HIGHEST
_N, _P, _R = 48, 17, 8
_B_TC = 3
_NQ = 4
_NI = _N // _NQ
_TW = _NI * _N * _P
_MW = _NI * _N * _R
_OW = _NI * _N


def _tc_body(trans_ref, tmask_ref, rules_ref, w0_ref, w1_ref, b0_ref, b1_ref,
             out_ref):
    _, N, _, P = trans_ref.shape
    R = tmask_ref.shape[-1]
    Nc = rules_ref.shape[-1]

    trans = trans_ref[0]
    th = trans.reshape(N * N, P)

    pgrid = jax.lax.broadcasted_iota(jnp.int32, (P, Nc), 0)
    oh0 = (pgrid == rules_ref[0:1, :]).astype(jnp.float32)
    oh1 = (pgrid == rules_ref[1:2, :]).astype(jnp.float32)

    rm = jnp.max(trans, axis=1)
    rmg = jnp.dot(rm, oh0, preferred_element_type=jnp.float32,
                  precision=_HIGH)
    t1 = jnp.dot(th, oh1, preferred_element_type=jnp.float32,
                 precision=_HIGH).reshape(N, N, Nc)
    scores = jnp.exp(rmg[:, None, :] + t1).reshape(N * N, Nc)

    cgrid = jax.lax.broadcasted_iota(jnp.int32, (Nc, R), 0)
    rgrid = jax.lax.broadcasted_iota(jnp.int32, (Nc, R), 1)
    W0 = jnp.zeros((Nc, R), jnp.float32)
    W1 = jnp.zeros((Nc, R), jnp.float32)
    for m in range(3):
        W0 = W0 + jnp.where(cgrid == 6 * rgrid + m, 1.0, 0.0) * w0_ref[m:m + 1, :]
        W1 = W1 + jnp.where(cgrid == 6 * rgrid + 3 + m, 1.0, 0.0) * w1_ref[m:m + 1, :]

    s0 = (jnp.dot(scores, W0, preferred_element_type=jnp.float32,
                  precision=_HIGH) + b0_ref[0:1, :]).reshape(N, N, R)
    s1 = (jnp.dot(scores, W1, preferred_element_type=jnp.float32,
                  precision=_HIGH) + b1_ref[0:1, :]).reshape(N, N, R)
    out_ref[0] = jnp.where(tmask_ref[0] == 0, s0, s1)


def _sc_body(trans_hbm, tmask_hbm, rrows_hbm, wrows_hbm, out_hbm,
             trans_v, tm_v, rules_v, wv_v, out_v, sem1, sem2):
    wid = lax.axis_index("s") * 2 + lax.axis_index("c")
    q = wid % _NQ
    rel = wid // _NQ

    c1 = pltpu.make_async_copy(trans_hbm.at[pl.ds(q * _TW, _TW)], trans_v, sem1)
    c1.start()
    c2 = pltpu.make_async_copy(tmask_hbm.at[pl.ds(q * _MW, _MW)], tm_v, sem2)
    c2.start()
    pltpu.sync_copy(rrows_hbm.at[wid], rules_v)
    pltpu.sync_copy(wrows_hbm.at[wid], wv_v)
    c1.wait()
    c2.wait()

    lane = lax.iota(jnp.int32, 16)
    lane17 = lane * 17
    lane8 = lane * 8

    w = [wv_v[k, :] for k in range(6)]
    bias0 = wv_v[6, :]
    bias1 = wv_v[7, :]
    pre0 = [lane17 + rules_v[2 * m, :] for m in range(6)]
    pre1 = [lane17 + rules_v[2 * m + 1, :] for m in range(6)]

    def body(i, carry):
        base_i = i * (_N * _P)
        tm_base = i * (_N * _R) + rel

        rms = []
        for m in range(6):
            v = plsc.load_gather(trans_v, [pre0[m] + base_i])
            for jb in range(1, 3):
                v = jnp.maximum(
                    v, plsc.load_gather(trans_v, [pre0[m] + (base_i + jb * 272)]))
            rms.append(jnp.max(v))

        for jb in range(3):
            off = base_i + jb * 272
            acc0 = bias0
            acc1 = bias1
            for m in range(3):
                t1v = plsc.load_gather(trans_v, [pre1[m] + off])
                acc0 = acc0 + w[m] * jnp.exp(t1v + rms[m])
            for m in range(3, 6):
                t1v = plsc.load_gather(trans_v, [pre1[m] + off])
                acc1 = acc1 + w[m] * jnp.exp(t1v + rms[m])
            tmv = plsc.load_gather(tm_v, [lane8 + (tm_base + jb * 128)])
            res = jnp.where(tmv == 0, acc0, acc1)
            out_v[pl.ds(i * _N + jb * 16, 16)] = res
        return carry

    lax.fori_loop(0, _NI, body, 0)
    pltpu.sync_copy(out_v, out_hbm.at[wid])


def _sc_call(trans3, tmask3, rrows, wrows):
    mesh = plsc.VectorSubcoreMesh(core_axis_name="c", subcore_axis_name="s")
    f = pl.kernel(
        _sc_body,
        out_type=jax.ShapeDtypeStruct((32, _OW), jnp.float32),
        mesh=mesh,
        compiler_params=pltpu.CompilerParams(needs_layout_passes=False),
        scratch_types=[
            pltpu.VMEM((_TW,), jnp.float32),
            pltpu.VMEM((_MW,), jnp.int32),
            pltpu.VMEM((12, 16), jnp.int32),
            pltpu.VMEM((8, 16), jnp.float32),
            pltpu.VMEM((_OW,), jnp.float32),
            pltpu.SemaphoreType.DMA,
            pltpu.SemaphoreType.DMA,
        ],
    )
    return f(trans3, tmask3, rrows, wrows)


def kernel(transitions, type_mask, rules, weights, biases, t_sections, c_sections):
    B, N, _, P = transitions.shape
    R = type_mask.shape[-1]

    rules_t = rules.T
    wflat = weights[:, :, 0]
    w0_mat = wflat[0::2, :].T
    w1_mat = wflat[1::2, :].T
    b0 = biases[0::2].reshape(1, R)
    b1 = biases[1::2].reshape(1, R)
    out4 = pl.pallas_call(
        _tc_body,
        grid=(_B_TC,),
        in_specs=[
            pl.BlockSpec((1, N, N, P), lambda b: (b, 0, 0, 0)),
            pl.BlockSpec((1, N, N, R), lambda b: (b, 0, 0, 0)),
            pl.BlockSpec((2, 48), lambda b: (0, 0)),
            pl.BlockSpec((3, R), lambda b: (0, 0)),
            pl.BlockSpec((3, R), lambda b: (0, 0)),
            pl.BlockSpec((1, R), lambda b: (0, 0)),
            pl.BlockSpec((1, R), lambda b: (0, 0)),
        ],
        out_specs=pl.BlockSpec((1, N, N, R), lambda b: (b, 0, 0, 0)),
        out_shape=jax.ShapeDtypeStruct((B, N, N, R), transitions.dtype),
        compiler_params=pltpu.CompilerParams(
            dimension_semantics=("arbitrary",)),
    )(transitions, type_mask, rules_t, w0_mat, w1_mat, b0, b1)

    trans3 = transitions[_B_TC].reshape(N * N * P)
    tmask3 = type_mask[_B_TC].reshape(N * N * R)
    rrows = jnp.broadcast_to(
        rules.reshape(R, 12, 1), (R, 12, 16)).astype(jnp.int32)
    rrows = jnp.repeat(rrows, _NQ, axis=0)
    wrow = jnp.concatenate(
        [wflat.reshape(R, 6), biases.reshape(R, 2)], axis=1)
    wrows = jnp.repeat(
        jnp.broadcast_to(wrow.reshape(R, 8, 1), (R, 8, 16)), _NQ, axis=0)
    sc_out = _sc_call(trans3, tmask3, rrows, wrows)
    sc_part = sc_out.reshape(R, _NQ, _NI, N).transpose(1, 2, 3, 0)
    sc_part = sc_part.reshape(1, N, N, R)

    return jax.lax.dynamic_update_slice(out4, sc_part, (_B_TC, 0, 0, 0))


# speedup vs baseline: 1.5707x; 1.5707x over previous
import jax
import jax.numpy as jnp
from jax.experimental import pallas as pl
from jax.experimental.pallas import tpu as pltpu

_HIGH = jax.lax.Precision.HIGHEST


def _tc_body(trans_ref, tmask_ref, rules_ref, w0_ref, w1_ref, b0_ref, b1_ref,
             out_ref):
    _, N, _, P = trans_ref.shape
    R = tmask_ref.shape[-1]
    Nc = rules_ref.shape[-1]

    trans = trans_ref[0]
    th = trans.reshape(N * N, P)

    pgrid = jax.lax.broadcasted_iota(jnp.int32, (P, Nc), 0)
    oh0 = (pgrid == rules_ref[0:1, :]).astype(jnp.float32)
    oh1 = (pgrid == rules_ref[1:2, :]).astype(jnp.float32)

    rm = jnp.max(trans, axis=1)
    rmg = jnp.dot(rm, oh0, preferred_element_type=jnp.float32,
                  precision=_HIGH)
    t1 = jnp.dot(th, oh1, preferred_element_type=jnp.float32,
                 precision=_HIGH).reshape(N, N, Nc)
    scores = jnp.exp(rmg[:, None, :] + t1).reshape(N * N, Nc)

    cgrid = jax.lax.broadcasted_iota(jnp.int32, (Nc, R), 0)
    rgrid = jax.lax.broadcasted_iota(jnp.int32, (Nc, R), 1)
    W0 = jnp.zeros((Nc, R), jnp.float32)
    W1 = jnp.zeros((Nc, R), jnp.float32)
    for m in range(3):
        W0 = W0 + jnp.where(cgrid == 6 * rgrid + m, 1.0, 0.0) * w0_ref[m:m + 1, :]
        W1 = W1 + jnp.where(cgrid == 6 * rgrid + 3 + m, 1.0, 0.0) * w1_ref[m:m + 1, :]

    s0 = (jnp.dot(scores, W0, preferred_element_type=jnp.float32,
                  precision=_HIGH) + b0_ref[0:1, :]).reshape(N, N, R)
    s1 = (jnp.dot(scores, W1, preferred_element_type=jnp.float32,
                  precision=_HIGH) + b1_ref[0:1, :]).reshape(N, N, R)
    out_ref[0] = jnp.where(tmask_ref[0] == 0, s0, s1)


def kernel(transitions, type_mask, rules, weights, biases, t_sections, c_sections):
    B, N, _, P = transitions.shape
    R = type_mask.shape[-1]
    rules_t = rules.T
    wflat = weights[:, :, 0]
    w0_mat = wflat[0::2, :].T
    w1_mat = wflat[1::2, :].T
    b0 = biases[0::2].reshape(1, R)
    b1 = biases[1::2].reshape(1, R)
    return pl.pallas_call(
        _tc_body,
        grid=(B,),
        in_specs=[
            pl.BlockSpec((1, N, N, P), lambda b: (b, 0, 0, 0)),
            pl.BlockSpec((1, N, N, R), lambda b: (b, 0, 0, 0)),
            pl.BlockSpec((2, 48), lambda b: (0, 0)),
            pl.BlockSpec((3, R), lambda b: (0, 0)),
            pl.BlockSpec((3, R), lambda b: (0, 0)),
            pl.BlockSpec((1, R), lambda b: (0, 0)),
            pl.BlockSpec((1, R), lambda b: (0, 0)),
        ],
        out_specs=pl.BlockSpec((1, N, N, R), lambda b: (b, 0, 0, 0)),
        out_shape=jax.ShapeDtypeStruct((B, N, N, R), transitions.dtype),
        compiler_params=pltpu.CompilerParams(
            dimension_semantics=("arbitrary",)),
    )(transitions, type_mask, rules_t, w0_mat, w1_mat, b0, b1)
